# baseline (device time: 26958 ns/iter reference)
import jax
import jax.numpy as jnp
from jax import lax
from jax.experimental import pallas as pl
from jax.experimental.pallas import tpu as pltpu

P = 32
GROUP = 8


def kernel(x, w_mat):
    m, k_per = x.shape
    k, n = w_mat.shape
    m_out = m // P

    def body(x_ref, w_ref, out_ref, xb_ref, xg_ref, wv_ref, send_sem,
             recv_sems, credit_sems, wdma_sem):
        me = lax.axis_index("i")

        wdma = pltpu.make_async_copy(w_ref, wv_ref, wdma_sem)
        wdma.start()

        barrier_sem = pltpu.get_barrier_semaphore()
        pl.semaphore_signal(barrier_sem, inc=1, device_id=(me,),
                            device_id_type=pl.DeviceIdType.MESH)
        pl.semaphore_wait(barrier_sem, 1)

        for t in range(1, P):
            pl.semaphore_signal(
                credit_sems.at[me], inc=1,
                device_id=((me + t) % P,),
                device_id_type=pl.DeviceIdType.MESH,
            )

        xb_ref[...] = x_ref[...].astype(jnp.bfloat16)
        xg_ref[me] = xb_ref[pl.ds(me * m_out, m_out), :]

        sends = []
        for t in range(1, P):
            tgt = (me + t) % P
            pl.semaphore_wait(credit_sems.at[tgt], 1)
            rdma = pltpu.make_async_remote_copy(
                src_ref=xb_ref.at[pl.ds(tgt * m_out, m_out), :],
                dst_ref=xg_ref.at[me],
                send_sem=send_sem,
                recv_sem=recv_sems.at[me],
                device_id=(tgt,),
                device_id_type=pl.DeviceIdType.MESH,
            )
            rdma.start()
            sends.append(rdma)

        wdma.wait()
        acc = jnp.dot(
            xg_ref[me].astype(jnp.float32),
            wv_ref[pl.ds(me * k_per, k_per), :],
            preferred_element_type=jnp.float32,
        )

        hops = list(range(1, P))
        for g0 in range(0, len(hops), GROUP):
            group = hops[g0:g0 + GROUP]
            for t in group:
                j = (me - t) % P
                recv = pltpu.make_async_remote_copy(
                    src_ref=xb_ref.at[pl.ds(j * m_out, m_out), :],
                    dst_ref=xg_ref.at[j],
                    send_sem=send_sem,
                    recv_sem=recv_sems.at[j],
                    device_id=(j,),
                    device_id_type=pl.DeviceIdType.MESH,
                )
                recv.wait_recv()
            for t in group:
                j = (me - t) % P
                acc = acc + jnp.dot(
                    xg_ref[j].astype(jnp.float32),
                    wv_ref[pl.ds(j * k_per, k_per), :],
                    preferred_element_type=jnp.float32,
                )

        c = 0.7978845608028654
        out_ref[...] = 0.5 * acc * (1.0 + jnp.tanh(c * (acc + 0.044715 * acc * acc * acc)))

        for rdma in sends:
            rdma.wait_send()

    return pl.pallas_call(
        body,
        out_shape=jax.ShapeDtypeStruct((m_out, n), jnp.float32),
        in_specs=[
            pl.BlockSpec(memory_space=pltpu.VMEM),
            pl.BlockSpec(memory_space=pltpu.HBM),
        ],
        out_specs=pl.BlockSpec(memory_space=pltpu.VMEM),
        scratch_shapes=[
            pltpu.VMEM((m, k_per), jnp.bfloat16),
            pltpu.VMEM((P, m_out, k_per), jnp.bfloat16),
            pltpu.VMEM((k, n), jnp.float32),
            pltpu.SemaphoreType.DMA,
            pltpu.SemaphoreType.DMA((P,)),
            pltpu.SemaphoreType.REGULAR((P,)),
            pltpu.SemaphoreType.DMA,
        ],
        compiler_params=pltpu.CompilerParams(collective_id=0),
    )(x, w_mat)


# device time: 21758 ns/iter; 1.2390x vs baseline; 1.2390x over previous
import jax
import jax.numpy as jnp
from jax import lax
from jax.experimental import pallas as pl
from jax.experimental.pallas import tpu as pltpu

P = 32
GROUP = 8


def kernel(x, w_mat):
    m, k_per = x.shape
    k, n = w_mat.shape
    m_out = m // P

    def body(x_ref, w_ref, out_ref, xb_ref, xg_ref, send_sem, recv_sems):
        me = lax.axis_index("i")

        barrier_sem = pltpu.get_barrier_semaphore()
        pl.semaphore_signal(barrier_sem, inc=1, device_id=(me,),
                            device_id_type=pl.DeviceIdType.MESH)
        pl.semaphore_wait(barrier_sem, 1)

        xb_ref[...] = x_ref[...].astype(jnp.bfloat16)
        xg_ref[me] = xb_ref[pl.ds(me * m_out, m_out), :]

        sends = []
        for t in range(1, P):
            tgt = (me + t) % P
            rdma = pltpu.make_async_remote_copy(
                src_ref=xb_ref.at[pl.ds(tgt * m_out, m_out), :],
                dst_ref=xg_ref.at[me],
                send_sem=send_sem,
                recv_sem=recv_sems.at[me],
                device_id=(tgt,),
                device_id_type=pl.DeviceIdType.MESH,
            )
            rdma.start()
            sends.append(rdma)

        acc = jnp.dot(
            xg_ref[me].astype(jnp.float32),
            w_ref[pl.ds(me * k_per, k_per), :],
            preferred_element_type=jnp.float32,
        )

        hops = list(range(1, P))
        groups = []
        g0 = 0
        for size in (8, 8, 8, 4, 2, 1):
            groups.append(hops[g0:g0 + size])
            g0 += size
        for group in groups:
            for t in group:
                j = (me - t) % P
                recv = pltpu.make_async_remote_copy(
                    src_ref=xb_ref.at[pl.ds(j * m_out, m_out), :],
                    dst_ref=xg_ref.at[j],
                    send_sem=send_sem,
                    recv_sem=recv_sems.at[j],
                    device_id=(j,),
                    device_id_type=pl.DeviceIdType.MESH,
                )
                recv.wait_recv()
            for t in group:
                j = (me - t) % P
                acc = acc + jnp.dot(
                    xg_ref[j].astype(jnp.float32),
                    w_ref[pl.ds(j * k_per, k_per), :],
                    preferred_element_type=jnp.float32,
                )

        c = 0.7978845608028654
        out_ref[...] = 0.5 * acc * (1.0 + jnp.tanh(c * (acc + 0.044715 * acc * acc * acc)))

        for rdma in sends:
            rdma.wait_send()

    return pl.pallas_call(
        body,
        out_shape=jax.ShapeDtypeStruct((m_out, n), jnp.float32),
        in_specs=[
            pl.BlockSpec(memory_space=pltpu.VMEM),
            pl.BlockSpec(memory_space=pltpu.VMEM),
        ],
        out_specs=pl.BlockSpec(memory_space=pltpu.VMEM),
        scratch_shapes=[
            pltpu.VMEM((m, k_per), jnp.bfloat16),
            pltpu.VMEM((P, m_out, k_per), jnp.bfloat16),
            pltpu.SemaphoreType.DMA,
            pltpu.SemaphoreType.DMA((P,)),
        ],
        compiler_params=pltpu.CompilerParams(collective_id=0),
    )(x, w_mat)
